# baseline (speedup 1.0000x reference)
import functools

import jax
import jax.numpy as jnp
from jax import lax
from jax.experimental import pallas as pl
from jax.experimental.pallas import tpu as pltpu
from jax.experimental.pallas import tpu_sc as plsc

_MAX_LEN = 8192
_PE_DIM = 128
_D_MODEL = 1024
_BATCH = 4
_OUT_D = _D_MODEL + _PE_DIM

_NW = 32
_ROWS_PER_W = _BATCH * _MAX_LEN // _NW
_CHUNK = 32
_NCHUNK = _ROWS_PER_W // _CHUNK


@functools.partial(
    pl.kernel,
    mesh=plsc.VectorSubcoreMesh(core_axis_name="c", subcore_axis_name="s"),
    out_type=jax.ShapeDtypeStruct((_BATCH * _MAX_LEN, _OUT_D), jnp.float32),
    scratch_types=[
        pltpu.VMEM((_CHUNK, _D_MODEL), jnp.float32),
        pltpu.VMEM((_CHUNK, _D_MODEL), jnp.float32),
        pltpu.VMEM((_CHUNK, _PE_DIM), jnp.float32),
        pltpu.VMEM((_CHUNK, _PE_DIM), jnp.float32),
        pltpu.SemaphoreType.DMA,
        pltpu.SemaphoreType.DMA,
    ],
)
def _sc_concat(x_hbm, pe_hbm, out_hbm, bx0, bx1, bp0, bp1, sem_i, sem_o):
    wid = lax.axis_index("s") * 2 + lax.axis_index("c")
    base = wid * _ROWS_PER_W
    t0 = base % _MAX_LEN
    bx = (bx0, bx1)
    bp = (bp0, bp1)

    def _in(i, slot):
        r = base + i * _CHUNK
        t = t0 + i * _CHUNK
        cx = pltpu.make_async_copy(x_hbm.at[pl.ds(r, _CHUNK), :], bx[slot],
                                   sem_i)
        cp = pltpu.make_async_copy(pe_hbm.at[pl.ds(t, _CHUNK), :], bp[slot],
                                   sem_i)
        cx.start()
        cp.start()
        return cx, cp

    def _out(i, slot):
        r = base + i * _CHUNK
        cx = pltpu.make_async_copy(
            bx[slot], out_hbm.at[pl.ds(r, _CHUNK), pl.ds(0, _D_MODEL)], sem_o)
        cp = pltpu.make_async_copy(
            bp[slot], out_hbm.at[pl.ds(r, _CHUNK), pl.ds(_D_MODEL, _PE_DIM)],
            sem_o)
        cx.start()
        cp.start()
        return cx, cp

    def _step(g):
        ins = [_in(g + s, s) for s in range(2)]
        outs = []
        for s in range(2):
            for c in ins[s]:
                c.wait()
            outs.append(_out(g + s, s))
        for s in range(2):
            for c in outs[s]:
                c.wait()

    pl.loop(0, _NCHUNK, step=2)(_step)


def kernel(x, pe_table):
    batch, max_len, d_model = x.shape
    x2 = x.reshape(batch * max_len, d_model)
    out = _sc_concat(x2, pe_table)
    return out.reshape(batch, max_len, _OUT_D)
